# baseline (speedup 1.0000x reference)
import jax
import jax.numpy as jnp
from jax.experimental import pallas as pl
from jax.experimental.pallas import tpu as pltpu

B = 256
N = 100
NP = 128
IN_DIM = 16
INNER = 256
LATENT = 128
BB = 32


def _relu(v):
    return jnp.maximum(v, 0.0)


def _gat_block(h, w_ref, asrc_ref, residual):
    wext = w_ref[...]
    dout = wext.shape[0] - 128
    asrc = asrc_ref[...]
    zext = jax.lax.dot_general(h, wext, (((1,), (1,)), ((), ())),
                               preferred_element_type=jnp.float32)
    z = zext[:, :dout]
    ed_all = zext[:, dout:dout + 1]
    colrow = jax.lax.broadcasted_iota(jnp.int32, (1, NP), 1)
    e_rows = []
    for b in range(BB):
        zb = z[b * NP:(b + 1) * NP, :]
        es = jax.lax.dot_general(asrc, zb, (((1,), (1,)), ((), ())),
                                 preferred_element_type=jnp.float32)
        es = jnp.where(colrow < N, es, -1e30)
        e_rows.append(ed_all[b * NP:(b + 1) * NP, :] + es)
    e = jnp.concatenate(e_rows, axis=0)
    e = jnp.where(e >= 0, e, 0.01 * e)
    m = jnp.max(e, axis=1, keepdims=True)
    p = jnp.exp(e - m)
    s = jnp.sum(p, axis=1, keepdims=True)
    a = p * jax.lax.reciprocal(s)
    outs = []
    for b in range(BB):
        zb = z[b * NP:(b + 1) * NP, :]
        ab = a[b * NP:(b + 1) * NP, :]
        outs.append(jnp.dot(ab, zb, preferred_element_type=jnp.float32))
    out = jnp.concatenate(outs, axis=0)
    if residual:
        out = _relu(out + h)
    return out


def _encoder_kernel(x_ref, wr_ref, b_ref,
                    w1_ref, as1_ref,
                    w2_ref, as2_ref,
                    w3_ref, as3_ref,
                    w4_ref, as4_ref,
                    out_ref):
    h = _relu(jax.lax.dot_general(x_ref[...], wr_ref[...],
                                  (((1,), (1,)), ((), ())),
                                  preferred_element_type=jnp.float32)
              + b_ref[...])
    h = _gat_block(h, w1_ref, as1_ref, True)
    h = _gat_block(h, w2_ref, as2_ref, True)
    h = _gat_block(h, w3_ref, as3_ref, True)
    out = _gat_block(h, w4_ref, as4_ref, False)
    for b in range(BB):
        out_ref[b, :, :] = out[b * NP:b * NP + N, :]


def kernel(x, W_remap, b_remap, Wfc1, Wa1, Wfc2, Wa2, Wfc3, Wa3, Wfc4, Wa4):
    xp = jnp.pad(x, ((0, 0), (0, NP - N), (0, 0))).reshape(B * NP, IN_DIM)
    br = b_remap.reshape(1, INNER)

    def prep(Wfc, Wa, dout):
        adst_row = jnp.pad(Wa[:, dout:] @ Wfc, ((0, 127), (0, 0)))
        return jnp.concatenate([Wfc, adst_row], axis=0), Wa[:, :dout]

    w1, as1 = prep(Wfc1, Wa1, INNER)
    w2, as2 = prep(Wfc2, Wa2, INNER)
    w3, as3 = prep(Wfc3, Wa3, INNER)
    w4, as4 = prep(Wfc4, Wa4, LATENT)

    full = lambda shp: pl.BlockSpec(shp, lambda i: (0, 0))
    out = pl.pallas_call(
        _encoder_kernel,
        grid=(B // BB,),
        in_specs=[
            pl.BlockSpec((BB * NP, IN_DIM), lambda i: (i, 0)),
            full((INNER, IN_DIM)), full((1, INNER)),
            full((INNER + 128, INNER)), full((1, INNER)),
            full((INNER + 128, INNER)), full((1, INNER)),
            full((INNER + 128, INNER)), full((1, INNER)),
            full((LATENT + 128, INNER)), full((1, LATENT)),
        ],
        out_specs=pl.BlockSpec((BB, N, LATENT), lambda i: (i, 0, 0)),
        out_shape=jax.ShapeDtypeStruct((B, N, LATENT), jnp.float32),
        compiler_params=pltpu.CompilerParams(
            dimension_semantics=("parallel",)),
    )(xp, W_remap, br, w1, as1, w2, as2, w3, as3, w4, as4)
    return out


# speedup vs baseline: 1.0597x; 1.0597x over previous
import jax
import jax.numpy as jnp
from jax.experimental import pallas as pl
from jax.experimental.pallas import tpu as pltpu

B = 256
N = 100
NP = 128
IN_DIM = 16
INNER = 256
LATENT = 128
BB = 32


def _relu(v):
    return jnp.maximum(v, 0.0)


def _gat_block(h, w_ref, p_mat, residual):
    wext = w_ref[...]
    dout = wext.shape[0] - 128
    zext = jax.lax.dot_general(h, wext, (((1,), (1,)), ((), ())),
                               preferred_element_type=jnp.float32)
    z = zext[:, :dout]
    ed_all = zext[:, dout:dout + 1]
    es_col = zext[:, dout + 1:dout + 2]
    es_mat = es_col.reshape(BB, NP)
    colrow = jax.lax.broadcasted_iota(jnp.int32, (BB, NP), 1)
    es_mat = jnp.where(colrow < N, es_mat, -1e30)
    e = ed_all + jnp.dot(p_mat, es_mat,
                         preferred_element_type=jnp.float32)
    e = jnp.where(e >= 0, e, 0.01 * e)
    m = jnp.max(e, axis=1, keepdims=True)
    p = jnp.exp(e - m)
    s = jnp.sum(p, axis=1, keepdims=True)
    a = p * jax.lax.reciprocal(s)
    outs = []
    for b in range(BB):
        zb = z[b * NP:(b + 1) * NP, :]
        ab = a[b * NP:(b + 1) * NP, :]
        outs.append(jnp.dot(ab, zb, preferred_element_type=jnp.float32))
    out = jnp.concatenate(outs, axis=0)
    if residual:
        out = _relu(out + h)
    return out


def _encoder_kernel(x_ref, wr_ref, b_ref,
                    w1_ref, w2_ref, w3_ref, w4_ref,
                    out_ref):
    r = jax.lax.broadcasted_iota(jnp.int32, (BB * NP, BB), 0) // NP
    c = jax.lax.broadcasted_iota(jnp.int32, (BB * NP, BB), 1)
    p_mat = (r == c).astype(jnp.float32)
    h = _relu(jax.lax.dot_general(x_ref[...], wr_ref[...],
                                  (((1,), (1,)), ((), ())),
                                  preferred_element_type=jnp.float32)
              + b_ref[...])
    h = _gat_block(h, w1_ref, p_mat, True)
    h = _gat_block(h, w2_ref, p_mat, True)
    h = _gat_block(h, w3_ref, p_mat, True)
    out = _gat_block(h, w4_ref, p_mat, False)
    for b in range(BB):
        out_ref[b, :, :] = out[b * NP:b * NP + N, :]


def kernel(x, W_remap, b_remap, Wfc1, Wa1, Wfc2, Wa2, Wfc3, Wa3, Wfc4, Wa4):
    xp = jnp.pad(x, ((0, 0), (0, NP - N), (0, 0))).reshape(B * NP, IN_DIM)
    br = b_remap.reshape(1, INNER)

    def prep(Wfc, Wa, dout):
        extra = jnp.concatenate([Wa[:, dout:] @ Wfc, Wa[:, :dout] @ Wfc], axis=0)
        return jnp.concatenate([Wfc, jnp.pad(extra, ((0, 126), (0, 0)))], axis=0)

    w1 = prep(Wfc1, Wa1, INNER)
    w2 = prep(Wfc2, Wa2, INNER)
    w3 = prep(Wfc3, Wa3, INNER)
    w4 = prep(Wfc4, Wa4, LATENT)

    full = lambda shp: pl.BlockSpec(shp, lambda i: (0, 0))
    out = pl.pallas_call(
        _encoder_kernel,
        grid=(B // BB,),
        in_specs=[
            pl.BlockSpec((BB * NP, IN_DIM), lambda i: (i, 0)),
            full((INNER, IN_DIM)), full((1, INNER)),
            full((INNER + 128, INNER)),
            full((INNER + 128, INNER)),
            full((INNER + 128, INNER)),
            full((LATENT + 128, INNER)),
        ],
        out_specs=pl.BlockSpec((BB, N, LATENT), lambda i: (i, 0, 0)),
        out_shape=jax.ShapeDtypeStruct((B, N, LATENT), jnp.float32),
        compiler_params=pltpu.CompilerParams(
            dimension_semantics=("parallel",)),
    )(xp, W_remap, br, w1, w2, w3, w4)
    return out


# speedup vs baseline: 1.1801x; 1.1136x over previous
import jax
import jax.numpy as jnp
from jax.experimental import pallas as pl
from jax.experimental.pallas import tpu as pltpu

B = 256
N = 100
NP = 104
IN_DIM = 16
INNER = 256
LATENT = 128
BB = 32


def _relu(v):
    return jnp.maximum(v, 0.0)


def _gat_block(h, w_ref, p_mat, residual):
    wext = w_ref[...]
    dout = wext.shape[0] - 128
    zext = jax.lax.dot_general(h, wext, (((1,), (1,)), ((), ())),
                               preferred_element_type=jnp.float32)
    z = zext[:, :dout]
    ed_all = zext[:, dout:dout + 1]
    es_col = zext[:, dout + 1:dout + 2]
    es_mat = es_col.reshape(BB, NP)
    colrow = jax.lax.broadcasted_iota(jnp.int32, (BB, NP), 1)
    es_mat = jnp.where(colrow < N, es_mat, -1e30)
    e = ed_all + jnp.dot(p_mat, es_mat,
                         preferred_element_type=jnp.float32)
    e = jnp.where(e >= 0, e, 0.01 * e)
    m = jnp.max(e, axis=1, keepdims=True)
    p = jnp.exp(e - m)
    s = jnp.sum(p, axis=1, keepdims=True)
    a = p * jax.lax.reciprocal(s)
    outs = []
    for b in range(BB):
        zb = z[b * NP:(b + 1) * NP, :]
        ab = a[b * NP:(b + 1) * NP, :]
        outs.append(jnp.dot(ab, zb, preferred_element_type=jnp.float32))
    out = jnp.concatenate(outs, axis=0)
    if residual:
        out = _relu(out + h)
    return out


def _encoder_kernel(x_ref, wr_ref, b_ref,
                    w1_ref, w2_ref, w3_ref, w4_ref,
                    out_ref):
    r = jax.lax.broadcasted_iota(jnp.int32, (BB * NP, BB), 0) // NP
    c = jax.lax.broadcasted_iota(jnp.int32, (BB * NP, BB), 1)
    p_mat = (r == c).astype(jnp.float32)
    h = _relu(jax.lax.dot_general(x_ref[...], wr_ref[...],
                                  (((1,), (1,)), ((), ())),
                                  preferred_element_type=jnp.float32)
              + b_ref[...])
    h = _gat_block(h, w1_ref, p_mat, True)
    h = _gat_block(h, w2_ref, p_mat, True)
    h = _gat_block(h, w3_ref, p_mat, True)
    out = _gat_block(h, w4_ref, p_mat, False)
    for b in range(BB):
        out_ref[b, :, :] = out[b * NP:b * NP + N, :]


def kernel(x, W_remap, b_remap, Wfc1, Wa1, Wfc2, Wa2, Wfc3, Wa3, Wfc4, Wa4):
    xp = jnp.pad(x, ((0, 0), (0, NP - N), (0, 0))).reshape(B * NP, IN_DIM)
    br = b_remap.reshape(1, INNER)

    def prep(Wfc, Wa, dout):
        extra = jnp.concatenate([Wa[:, dout:] @ Wfc, Wa[:, :dout] @ Wfc], axis=0)
        return jnp.concatenate([Wfc, jnp.pad(extra, ((0, 126), (0, 0)))], axis=0)

    w1 = prep(Wfc1, Wa1, INNER)
    w2 = prep(Wfc2, Wa2, INNER)
    w3 = prep(Wfc3, Wa3, INNER)
    w4 = prep(Wfc4, Wa4, LATENT)

    full = lambda shp: pl.BlockSpec(shp, lambda i: (0, 0))
    out = pl.pallas_call(
        _encoder_kernel,
        grid=(B // BB,),
        in_specs=[
            pl.BlockSpec((BB * NP, IN_DIM), lambda i: (i, 0)),
            full((INNER, IN_DIM)), full((1, INNER)),
            full((INNER + 128, INNER)),
            full((INNER + 128, INNER)),
            full((INNER + 128, INNER)),
            full((LATENT + 128, INNER)),
        ],
        out_specs=pl.BlockSpec((BB, N, LATENT), lambda i: (i, 0, 0)),
        out_shape=jax.ShapeDtypeStruct((B, N, LATENT), jnp.float32),
        compiler_params=pltpu.CompilerParams(
            dimension_semantics=("parallel",)),
    )(xp, W_remap, br, w1, w2, w3, w4)
    return out


# speedup vs baseline: 1.2274x; 1.0401x over previous
import jax
import jax.numpy as jnp
from jax.experimental import pallas as pl
from jax.experimental.pallas import tpu as pltpu

B = 256
N = 100
NP = 104
IN_DIM = 16
INNER = 256
LATENT = 128
BB = 64


def _relu(v):
    return jnp.maximum(v, 0.0)


def _gat_block(h, w_ref, p_mat, residual):
    wext = w_ref[...]
    dout = wext.shape[0] - 128
    zext = jax.lax.dot_general(h, wext, (((1,), (1,)), ((), ())),
                               preferred_element_type=jnp.float32)
    z = zext[:, :dout]
    ed_all = zext[:, dout:dout + 1]
    es_col = zext[:, dout + 1:dout + 2]
    es_mat = es_col.reshape(BB, NP)
    colrow = jax.lax.broadcasted_iota(jnp.int32, (BB, NP), 1)
    es_mat = jnp.where(colrow < N, es_mat, -1e30)
    e = ed_all + jnp.dot(p_mat, es_mat,
                         preferred_element_type=jnp.float32)
    e = jnp.where(e >= 0, e, 0.01 * e)
    m = jnp.max(e, axis=1, keepdims=True)
    p = jnp.exp(e - m)
    s = jnp.sum(p, axis=1, keepdims=True)
    a = p * jax.lax.reciprocal(s)
    outs = []
    for b in range(BB):
        zb = z[b * NP:(b + 1) * NP, :]
        ab = a[b * NP:(b + 1) * NP, :]
        outs.append(jnp.dot(ab, zb, preferred_element_type=jnp.float32))
    out = jnp.concatenate(outs, axis=0)
    if residual:
        out = _relu(out + h)
    return out


def _encoder_kernel(x_ref, wr_ref, b_ref,
                    w1_ref, w2_ref, w3_ref, w4_ref,
                    out_ref):
    r = jax.lax.broadcasted_iota(jnp.int32, (BB * NP, BB), 0) // NP
    c = jax.lax.broadcasted_iota(jnp.int32, (BB * NP, BB), 1)
    p_mat = (r == c).astype(jnp.float32)
    h = _relu(jax.lax.dot_general(x_ref[...], wr_ref[...],
                                  (((1,), (1,)), ((), ())),
                                  preferred_element_type=jnp.float32)
              + b_ref[...])
    h = _gat_block(h, w1_ref, p_mat, True)
    h = _gat_block(h, w2_ref, p_mat, True)
    h = _gat_block(h, w3_ref, p_mat, True)
    out = _gat_block(h, w4_ref, p_mat, False)
    for b in range(BB):
        out_ref[b, :, :] = out[b * NP:b * NP + N, :]


def kernel(x, W_remap, b_remap, Wfc1, Wa1, Wfc2, Wa2, Wfc3, Wa3, Wfc4, Wa4):
    xp = jnp.pad(x, ((0, 0), (0, NP - N), (0, 0))).reshape(B * NP, IN_DIM)
    br = b_remap.reshape(1, INNER)

    def prep(Wfc, Wa, dout):
        extra = jnp.concatenate([Wa[:, dout:] @ Wfc, Wa[:, :dout] @ Wfc], axis=0)
        return jnp.concatenate([Wfc, jnp.pad(extra, ((0, 126), (0, 0)))], axis=0)

    w1 = prep(Wfc1, Wa1, INNER)
    w2 = prep(Wfc2, Wa2, INNER)
    w3 = prep(Wfc3, Wa3, INNER)
    w4 = prep(Wfc4, Wa4, LATENT)

    full = lambda shp: pl.BlockSpec(shp, lambda i: (0, 0))
    out = pl.pallas_call(
        _encoder_kernel,
        grid=(B // BB,),
        in_specs=[
            pl.BlockSpec((BB * NP, IN_DIM), lambda i: (i, 0)),
            full((INNER, IN_DIM)), full((1, INNER)),
            full((INNER + 128, INNER)),
            full((INNER + 128, INNER)),
            full((INNER + 128, INNER)),
            full((LATENT + 128, INNER)),
        ],
        out_specs=pl.BlockSpec((BB, N, LATENT), lambda i: (i, 0, 0)),
        out_shape=jax.ShapeDtypeStruct((B, N, LATENT), jnp.float32),
        compiler_params=pltpu.CompilerParams(
            dimension_semantics=("parallel",)),
    )(xp, W_remap, br, w1, w2, w3, w4)
    return out
